# baseline (speedup 1.0000x reference)
import functools

import jax
import jax.numpy as jnp
from jax import lax
from jax.experimental import pallas as pl
from jax.experimental.pallas import tpu as pltpu
from jax.experimental.pallas import tpu_sc as plsc

B = 1024
DENSE_DIM = 13
SPARSE_DIM = 26
VOCAB = 1000
EMB = 64
K = 64
FN = DENSE_DIM + SPARSE_DIM * EMB

NC = 2
NS = 16
NW = NC * NS
TOTAL_ROWS = B * SPARSE_DIM
ROWS_PER_W = TOTAL_ROWS // NW
CHUNK = 104
NCHUNK = ROWS_PER_W // CHUNK


def _sc_gather(table_pad, idx_flat):
  mesh = plsc.VectorSubcoreMesh(core_axis_name="c", subcore_axis_name="s")

  @functools.partial(
      pl.kernel,
      mesh=mesh,
      out_type=jax.ShapeDtypeStruct((TOTAL_ROWS, 2 * EMB), jnp.float32),
      scratch_types=[
          pltpu.VMEM((ROWS_PER_W,), jnp.int32),
          pltpu.VMEM((ROWS_PER_W, 2 * EMB), jnp.float32),
          pltpu.SemaphoreType.DMA,
      ],
  )
  def k(table_hbm, idx_hbm, out_hbm, idx_v, rows_v, sem):
    wid = lax.axis_index("s") * NC + lax.axis_index("c")
    pltpu.sync_copy(idx_hbm.at[pl.ds(wid * ROWS_PER_W, ROWS_PER_W)], idx_v)
    copies = []
    for j in range(NCHUNK):
      copies.append(
          pltpu.async_copy(
              table_hbm.at[idx_v.at[pl.ds(j * CHUNK, CHUNK)]],
              rows_v.at[pl.ds(j * CHUNK, CHUNK)],
              sem,
          ))
    for c in copies:
      c.wait()
    base = wid * ROWS_PER_W
    pltpu.sync_copy(rows_v, out_hbm.at[pl.ds(base, ROWS_PER_W)])

  return k(table_pad, idx_flat)


def _prep_body(t_ref, out_ref):
  y = jnp.transpose(t_ref[0], (1, 0))
  out_ref[0, :, :EMB] = y
  out_ref[0, :, EMB:] = jnp.zeros((VOCAB, EMB), jnp.float32)


def _tc_prep_table(tables_t):
  out = pl.pallas_call(
      _prep_body,
      grid=(SPARSE_DIM,),
      in_specs=[pl.BlockSpec((1, EMB, VOCAB), lambda f: (f, 0, 0))],
      out_specs=pl.BlockSpec((1, VOCAB, 2 * EMB), lambda f: (f, 0, 0)),
      out_shape=jax.ShapeDtypeStruct((SPARSE_DIM, VOCAB, 2 * EMB),
                                     jnp.float32),
  )(tables_t)
  return out.reshape(SPARSE_DIM * VOCAB, 2 * EMB)


def _tc_body(dense_ref, emb_ref, w0_ref, w_ref, Vt_ref, W1_ref, b1_ref,
             W2_ref, b2_ref, W3_ref, b3_ref, Wo_ref, bo_ref, out_ref):
  pieces = [dense_ref[...]]
  for f in range(SPARSE_DIM):
    pieces.append(emb_ref[pl.ds(f * B, B), :EMB])
  x = jnp.concatenate(pieces, axis=1)
  nt = (((1,), (1,)), ((), ()))
  linear = jnp.dot(x, w_ref[...], preferred_element_type=jnp.float32)
  linear = linear + w0_ref[0, 0]
  xv = lax.dot_general(x, Vt_ref[...], nt, preferred_element_type=jnp.float32)
  x2v2 = lax.dot_general(jnp.square(x), jnp.square(Vt_ref[...]), nt,
                         preferred_element_type=jnp.float32)
  inter = 0.5 * jnp.sum(jnp.square(xv) - x2v2, axis=1, keepdims=True)
  fm = linear + inter
  h = jnp.dot(x, W1_ref[...], preferred_element_type=jnp.float32)
  h = jnp.maximum(h + b1_ref[...], 0.0)
  h = jnp.dot(h, W2_ref[...], preferred_element_type=jnp.float32)
  h = jnp.maximum(h + b2_ref[...], 0.0)
  h = jnp.dot(h, W3_ref[...], preferred_element_type=jnp.float32)
  h = jnp.maximum(h + b3_ref[...], 0.0)
  deep = jnp.dot(h, Wo_ref[...], preferred_element_type=jnp.float32)
  deep = deep + bo_ref[0, 0]
  out_ref[...] = jax.nn.sigmoid(0.5 * (fm + deep))


def kernel(inputs, tables, w0, w, V, W1, b1, W2, b2, W3, b3, Wo, bo):
  dense = inputs[:, :DENSE_DIM]
  idx = inputs[:, DENSE_DIM:].astype(jnp.int32)
  idx_flat = (idx.T + jnp.arange(SPARSE_DIM, dtype=jnp.int32)[:, None] * VOCAB
              ).reshape(TOTAL_ROWS)
  table_pad = _tc_prep_table(jnp.transpose(tables, (0, 2, 1)))

  emb2 = _sc_gather(table_pad, idx_flat)

  out = pl.pallas_call(
      _tc_body,
      out_shape=jax.ShapeDtypeStruct((B, 1), jnp.float32),
  )(dense, emb2, w0.reshape(1, 1), w, V.T, W1, b1.reshape(1, 1024),
    W2, b2.reshape(1, 512), W3, b3.reshape(1, 256), Wo, bo.reshape(1, 1))
  return out
